# baseline (speedup 1.0000x reference)
import functools

import jax
import jax.numpy as jnp
from jax import lax
from jax.experimental import pallas as pl
from jax.experimental.pallas import tpu as pltpu
from jax.experimental.pallas import tpu_sc as plsc

NNODE = 10000
NP = 10240
NEDGE = 160000
CHUNK = 40
NC, NS = 2, 16
RP = NP // NS

_MESH = plsc.VectorSubcoreMesh(core_axis_name="c", subcore_axis_name="s")
_SC_PARAMS = pltpu.CompilerParams(use_tc_tiling_on_sc=False)


_DEG_ROWS = NEDGE // (NC * NS) // CHUNK


def _deg_body(e_h, ones_h, zeros_h, outp_h, inp_h,
              sidx2, didx2, ones_v, shout, shin, sem_a, sem_b):
    c = lax.axis_index("c")
    s = lax.axis_index("s")
    rbase = (c * NS + s) * _DEG_ROWS
    pltpu.sync_copy(e_h.at[0, pl.ds(rbase, _DEG_ROWS)], sidx2)
    pltpu.sync_copy(e_h.at[1, pl.ds(rbase, _DEG_ROWS)], didx2)
    pltpu.sync_copy(ones_h, ones_v)
    rows = pl.ds(s * RP, RP)
    pltpu.sync_copy(zeros_h, shout.at[rows])
    pltpu.sync_copy(zeros_h, shin.at[rows])
    plsc.subcore_barrier()

    @pl.loop(0, _DEG_ROWS, step=5)
    def _grp(i):
        hs = []
        for k in range(5):
            hs.append(pltpu.async_copy(ones_v, shout.at[sidx2.at[i + k]],
                                       sem_a, add=True))
            hs.append(pltpu.async_copy(ones_v, shin.at[didx2.at[i + k]],
                                       sem_b, add=True))
        for h in hs:
            h.wait()

    plsc.subcore_barrier()
    pltpu.sync_copy(shout.at[rows], outp_h.at[c, rows])
    pltpu.sync_copy(shin.at[rows], inp_h.at[c, rows])


_deg_call = functools.partial(
    pl.kernel,
    out_type=[jax.ShapeDtypeStruct((NC, NP, 16), jnp.float32),
              jax.ShapeDtypeStruct((NC, NP, 16), jnp.float32)],
    mesh=_MESH,
    compiler_params=_SC_PARAMS,
    scratch_types=[
        pltpu.VMEM((_DEG_ROWS, CHUNK), jnp.int32),
        pltpu.VMEM((_DEG_ROWS, CHUNK), jnp.int32),
        pltpu.VMEM((CHUNK, 16), jnp.float32),
        pltpu.VMEM_SHARED((NP, 16), jnp.float32),
        pltpu.VMEM_SHARED((NP, 16), jnp.float32),
        pltpu.SemaphoreType.DMA,
        pltpu.SemaphoreType.DMA,
    ],
)(_deg_body)


def _make_agg(passes, per_sc_edges_split, nbuf, chunk=CHUNK):
    if per_sc_edges_split:
        nrows = NEDGE // (NC * NS) // chunk
    else:
        nrows = NEDGE // NS // chunk
    nout = NC if per_sc_edges_split else passes * NC

    def body(e_h, h_h, zeros_h, agg_h, sidx2, didx2, *rest):
        bufs = rest[:nbuf]
        table = rest[nbuf]
        acc = rest[nbuf + 1]
        gsem = rest[nbuf + 2:2 * nbuf + 2]
        ssem = rest[2 * nbuf + 2:3 * nbuf + 2]
        c = lax.axis_index("c")
        s = lax.axis_index("s")
        if per_sc_edges_split:
            rbase = (c * NS + s) * nrows
        else:
            rbase = s * nrows
        rows = pl.ds(s * RP, RP)
        pltpu.sync_copy(e_h.at[0, pl.ds(rbase, nrows)], sidx2)
        pltpu.sync_copy(e_h.at[1, pl.ds(rbase, nrows)], didx2)

        for q in range(passes):
            qidx = 0 if per_sc_edges_split else c * passes + q
            pltpu.sync_copy(h_h.at[qidx, rows], table.at[rows])
            pltpu.sync_copy(zeros_h, acc.at[rows])
            plsc.subcore_barrier()

            @pl.loop(0, nrows, step=nbuf)
            def _grp(i):
                hg = [pltpu.async_copy(table.at[sidx2.at[i + k]], bufs[k],
                                       gsem[k]) for k in range(nbuf)]
                hs = []
                for k in range(nbuf):
                    hg[k].wait()
                    hs.append(pltpu.async_copy(bufs[k],
                                               acc.at[didx2.at[i + k]],
                                               ssem[k], add=True))
                for h in hs:
                    h.wait()

            plsc.subcore_barrier()
            out_idx = c if per_sc_edges_split else c * passes + q
            pltpu.sync_copy(acc.at[rows], agg_h.at[out_idx, rows])

    return functools.partial(
        pl.kernel,
        out_type=jax.ShapeDtypeStruct((nout, NP, 64), jnp.float32),
        mesh=_MESH,
        compiler_params=_SC_PARAMS,
        scratch_types=(
            [pltpu.VMEM((nrows, chunk), jnp.int32),
             pltpu.VMEM((nrows, chunk), jnp.int32)]
            + [pltpu.VMEM((chunk, 64), jnp.float32)] * nbuf
            + [pltpu.VMEM_SHARED((NP, 64), jnp.float32),
               pltpu.VMEM_SHARED((NP, 64), jnp.float32)]
            + [pltpu.SemaphoreType.DMA] * (2 * nbuf)
        ),
    )(body)


_agg1_call = _make_agg(passes=2, per_sc_edges_split=False, nbuf=5, chunk=80)
_agg2_call = _make_agg(passes=1, per_sc_edges_split=True, nbuf=5)


_PRED_NBUF = 5
_PRED_ROWS = NEDGE // (NC * NS) // CHUNK


def _pred_body(pe_h, ne_h, a_h, b_h, w_h, sp_h, sn_h, uidx2, vidx2, *rest):
    c = lax.axis_index("c")
    s = lax.axis_index("s")
    nbuf = _PRED_NBUF
    bufs = rest[:nbuf]
    obufs = rest[nbuf:2 * nbuf]
    wv = rest[2 * nbuf]
    sha = rest[2 * nbuf + 1]
    shb = rest[2 * nbuf + 2]
    asem = rest[2 * nbuf + 3:3 * nbuf + 3]
    bsem = rest[3 * nbuf + 3:4 * nbuf + 3]
    osem = rest[4 * nbuf + 3:5 * nbuf + 3]
    rbase = (c * NS + s) * _PRED_ROWS
    rows = pl.ds(s * RP, RP)
    pltpu.sync_copy(a_h.at[rows], sha.at[rows])
    pltpu.sync_copy(b_h.at[rows], shb.at[rows])
    pltpu.sync_copy(w_h, wv)
    w0 = wv[pl.ds(0, 16)]
    w1 = wv[pl.ds(16, 16)]
    w2 = wv[pl.ds(32, 16)]
    w3 = wv[pl.ds(48, 16)]
    bp2s = wv[pl.ds(64, 16)][0]
    lane = lax.iota(jnp.int32, 16)
    plsc.subcore_barrier()

    for e_h, o_h in ((pe_h, sp_h), (ne_h, sn_h)):
        pltpu.sync_copy(e_h.at[0, pl.ds(rbase, _PRED_ROWS)], uidx2)
        pltpu.sync_copy(e_h.at[1, pl.ds(rbase, _PRED_ROWS)], vidx2)

        @pl.loop(0, _PRED_ROWS, step=nbuf)
        def _grp(i):
            ha = [pltpu.async_copy(sha.at[uidx2.at[i + k]],
                                   bufs[k].at[pl.ds(0, CHUNK)], asem[k])
                  for k in range(nbuf)]
            hb = []
            for k in range(nbuf):
                ha[k].wait()
                hb.append(pltpu.async_copy(shb.at[vidx2.at[i + k]],
                                           bufs[k].at[pl.ds(0, CHUNK)],
                                           bsem[k], add=True))
            ho = []
            for k in range(nbuf):
                hb[k].wait()
                buf = bufs[k]
                obuf = obufs[k]

                @pl.loop(0, 3)
                def _egrp(g):
                    z = jnp.float32(0.0)
                    svec = jnp.zeros((16,), jnp.float32)
                    for j in range(16):
                        e = g * 16 + j
                        acc = (jnp.maximum(buf[e, 0:16], z) * w0
                               + jnp.maximum(buf[e, 16:32], z) * w1
                               + jnp.maximum(buf[e, 32:48], z) * w2
                               + jnp.maximum(buf[e, 48:64], z) * w3)
                        sval = jnp.sum(acc) + bp2s
                        svec = jnp.where(lane == j, sval, svec)
                    obuf[pl.ds(g * 16, 16)] = svec

                ho.append(pltpu.async_copy(
                    obuf.at[pl.ds(0, CHUNK)],
                    o_h.at[pl.ds((rbase + i + k) * CHUNK, CHUNK)],
                    osem[k]))
            for h in ho:
                h.wait()


_pred_call = functools.partial(
    pl.kernel,
    out_type=[jax.ShapeDtypeStruct((NEDGE,), jnp.float32),
              jax.ShapeDtypeStruct((NEDGE,), jnp.float32)],
    mesh=_MESH,
    compiler_params=pltpu.CompilerParams(use_tc_tiling_on_sc=False,
                                         needs_layout_passes=False),
    scratch_types=(
        [pltpu.VMEM((_PRED_ROWS, CHUNK), jnp.int32),
         pltpu.VMEM((_PRED_ROWS, CHUNK), jnp.int32)]
        + [pltpu.VMEM((48, 64), jnp.float32)] * _PRED_NBUF
        + [pltpu.VMEM((48,), jnp.float32)] * _PRED_NBUF
        + [pltpu.VMEM((80,), jnp.float32),
           pltpu.VMEM_SHARED((NP, 64), jnp.float32),
           pltpu.VMEM_SHARED((NP, 64), jnp.float32)]
        + [pltpu.SemaphoreType.DMA] * (3 * _PRED_NBUF)
    ),
)(_pred_body)


_RB = 1000


def _rsqrt_deg(ref):
    d = ref[0, :, 0:1] + ref[1, :, 0:1]
    return lax.rsqrt(jnp.maximum(d, 1.0))


def _k2_body(x_ref, w_ref, dego_ref, out_ref):
    ns = _rsqrt_deg(dego_ref)
    xw = jnp.dot(x_ref[...].astype(jnp.bfloat16),
                 w_ref[0].astype(jnp.bfloat16),
                 preferred_element_type=jnp.float32)
    out_ref[0] = xw * ns


def _k2(x, W1q, degout_p):
    return pl.pallas_call(
        _k2_body,
        grid=(10, 4),
        in_specs=[
            pl.BlockSpec((_RB, 512), lambda i, h: (i, 0)),
            pl.BlockSpec((1, 512, 64), lambda i, h: (h, 0, 0)),
            pl.BlockSpec((2, _RB, 16), lambda i, h: (0, i, 0)),
        ],
        out_specs=pl.BlockSpec((1, _RB, 64), lambda i, h: (h, i, 0)),
        out_shape=jax.ShapeDtypeStruct((4, NP, 64), jnp.float32),
    )(x, W1q, degout_p)


def _k4_body(agg_ref, dego_ref, degi_ref, b1_ref, w2_ref, out_ref):
    ns = _rsqrt_deg(dego_ref)
    nd = _rsqrt_deg(degi_ref)
    t = None
    for q in range(4):
        a = jnp.maximum(agg_ref[q] * nd + b1_ref[q:q + 1, :], 0.0)
        aq = jnp.dot(a, w2_ref[64 * q:64 * q + 64],
                     preferred_element_type=jnp.float32)
        t = aq if t is None else t + aq
    out_ref[...] = t * ns


def _k4(agg1_p, degout_p, degin_p, b1r, W2):
    return pl.pallas_call(
        _k4_body,
        grid=(10,),
        in_specs=[
            pl.BlockSpec((4, _RB, 64), lambda i: (0, i, 0)),
            pl.BlockSpec((2, _RB, 16), lambda i: (0, i, 0)),
            pl.BlockSpec((2, _RB, 16), lambda i: (0, i, 0)),
            pl.BlockSpec((4, 64), lambda i: (0, 0)),
            pl.BlockSpec((256, 64), lambda i: (0, 0)),
        ],
        out_specs=pl.BlockSpec((_RB, 64), lambda i: (i, 0)),
        out_shape=jax.ShapeDtypeStruct((NP, 64), jnp.float32),
    )(agg1_p, degout_p, degin_p, b1r, W2)


def _k6_body(agg_ref, degi_ref, b2_ref, wp1_ref, bp1_ref, a_ref, b_ref):
    nd = _rsqrt_deg(degi_ref)
    h2 = (agg_ref[0] + agg_ref[1]) * nd + b2_ref[0:1, :]
    a_ref[...] = (jnp.dot(h2, wp1_ref[0:64], preferred_element_type=jnp.float32)
                  + bp1_ref[0:1, :])
    b_ref[...] = jnp.dot(h2, wp1_ref[64:128], preferred_element_type=jnp.float32)


def _k6(agg2_p, degin_p, b2r, Wp1, bp1r):
    return pl.pallas_call(
        _k6_body,
        grid=(10,),
        in_specs=[
            pl.BlockSpec((2, _RB, 64), lambda i: (0, i, 0)),
            pl.BlockSpec((2, _RB, 16), lambda i: (0, i, 0)),
            pl.BlockSpec((1, 64), lambda i: (0, 0)),
            pl.BlockSpec((128, 64), lambda i: (0, 0)),
            pl.BlockSpec((1, 64), lambda i: (0, 0)),
        ],
        out_specs=[
            pl.BlockSpec((_RB, 64), lambda i: (i, 0)),
            pl.BlockSpec((_RB, 64), lambda i: (i, 0)),
        ],
        out_shape=[jax.ShapeDtypeStruct((NP, 64), jnp.float32),
                   jax.ShapeDtypeStruct((NP, 64), jnp.float32)],
    )(agg2_p, degin_p, b2r, Wp1, bp1r)


def kernel(x, edge_index, pos_edge_index, neg_edge_index,
           W1, b1, W2, b2, Wp1, bp1, Wp2, bp2):
    ones16 = jnp.ones((CHUNK, 16), jnp.float32)
    zeros16 = jnp.zeros((RP, 16), jnp.float32)
    zeros64 = jnp.zeros((RP, 64), jnp.float32)

    e3 = edge_index.reshape(2, NEDGE // CHUNK, CHUNK)
    p3 = pos_edge_index.reshape(2, NEDGE // CHUNK, CHUNK)
    n3 = neg_edge_index.reshape(2, NEDGE // CHUNK, CHUNK)

    degout_p, degin_p = _deg_call(e3, ones16, zeros16)

    w1q = W1.reshape(512, 4, 64).transpose(1, 0, 2)
    h1s = _k2(x, w1q, degout_p)
    e80 = edge_index.reshape(2, NEDGE // 80, 80)
    agg1_p = _agg1_call(e80, h1s, zeros64)

    h2in = _k4(agg1_p, degout_p, degin_p, b1.reshape(4, 64), W2)
    agg2_p = _agg2_call(e3, h2in.reshape(1, NP, 64), zeros64)

    A, B = _k6(agg2_p, degin_p, b2.reshape(1, 64), Wp1, bp1.reshape(1, 64))
    wvec = jnp.concatenate([Wp2.reshape(64), bp2, jnp.zeros((15,), jnp.float32)])
    pos, neg = _pred_call(p3, n3, A, B, wvec)
    return (pos, neg)


# speedup vs baseline: 1.0278x; 1.0278x over previous
import functools

import jax
import jax.numpy as jnp
from jax import lax
from jax.experimental import pallas as pl
from jax.experimental.pallas import tpu as pltpu
from jax.experimental.pallas import tpu_sc as plsc

NNODE = 10000
NP = 10240
NEDGE = 160000
CHUNK = 40
NC, NS = 2, 16
RP = NP // NS

_MESH = plsc.VectorSubcoreMesh(core_axis_name="c", subcore_axis_name="s")
_SC_PARAMS = pltpu.CompilerParams(use_tc_tiling_on_sc=False)


_DEG_ROWS = NEDGE // (NC * NS) // CHUNK


def _deg_body(e_h, ones_h, zeros_h, outp_h, inp_h,
              sidx2, didx2, ones_v, shout, shin, sem_a, sem_b):
    c = lax.axis_index("c")
    s = lax.axis_index("s")
    rbase = (c * NS + s) * _DEG_ROWS
    pltpu.sync_copy(e_h.at[0, pl.ds(rbase, _DEG_ROWS)], sidx2)
    pltpu.sync_copy(e_h.at[1, pl.ds(rbase, _DEG_ROWS)], didx2)
    pltpu.sync_copy(ones_h, ones_v)
    rows = pl.ds(s * RP, RP)
    pltpu.sync_copy(zeros_h, shout.at[rows])
    pltpu.sync_copy(zeros_h, shin.at[rows])
    plsc.subcore_barrier()

    @pl.loop(0, _DEG_ROWS, step=5)
    def _grp(i):
        hs = []
        for k in range(5):
            hs.append(pltpu.async_copy(ones_v, shout.at[sidx2.at[i + k]],
                                       sem_a, add=True))
            hs.append(pltpu.async_copy(ones_v, shin.at[didx2.at[i + k]],
                                       sem_b, add=True))
        for h in hs:
            h.wait()

    plsc.subcore_barrier()
    pltpu.sync_copy(shout.at[rows], outp_h.at[c, rows])
    pltpu.sync_copy(shin.at[rows], inp_h.at[c, rows])


_deg_call = functools.partial(
    pl.kernel,
    out_type=[jax.ShapeDtypeStruct((NC, NP, 16), jnp.float32),
              jax.ShapeDtypeStruct((NC, NP, 16), jnp.float32)],
    mesh=_MESH,
    compiler_params=_SC_PARAMS,
    scratch_types=[
        pltpu.VMEM((_DEG_ROWS, CHUNK), jnp.int32),
        pltpu.VMEM((_DEG_ROWS, CHUNK), jnp.int32),
        pltpu.VMEM((CHUNK, 16), jnp.float32),
        pltpu.VMEM_SHARED((NP, 16), jnp.float32),
        pltpu.VMEM_SHARED((NP, 16), jnp.float32),
        pltpu.SemaphoreType.DMA,
        pltpu.SemaphoreType.DMA,
    ],
)(_deg_body)


def _make_agg(passes, per_sc_edges_split, nbuf, chunk=CHUNK):
    if per_sc_edges_split:
        nrows = NEDGE // (NC * NS) // chunk
    else:
        nrows = NEDGE // NS // chunk
    nout = NC if per_sc_edges_split else passes * NC

    def body(e_h, h_h, zeros_h, agg_h, sidx2, didx2, *rest):
        bufs = rest[:nbuf]
        table = rest[nbuf]
        acc = rest[nbuf + 1]
        gsem = rest[nbuf + 2:2 * nbuf + 2]
        ssem = rest[2 * nbuf + 2:3 * nbuf + 2]
        c = lax.axis_index("c")
        s = lax.axis_index("s")
        if per_sc_edges_split:
            rbase = (c * NS + s) * nrows
        else:
            rbase = s * nrows
        rows = pl.ds(s * RP, RP)
        pltpu.sync_copy(e_h.at[0, pl.ds(rbase, nrows)], sidx2)
        pltpu.sync_copy(e_h.at[1, pl.ds(rbase, nrows)], didx2)

        for q in range(passes):
            qidx = 0 if per_sc_edges_split else c * passes + q
            pltpu.sync_copy(h_h.at[qidx, rows], table.at[rows])
            pltpu.sync_copy(zeros_h, acc.at[rows])
            plsc.subcore_barrier()

            @pl.loop(0, nrows, step=nbuf)
            def _grp(i):
                hg = [pltpu.async_copy(table.at[sidx2.at[i + k]], bufs[k],
                                       gsem[k]) for k in range(nbuf)]
                hs = []
                for k in range(nbuf):
                    hg[k].wait()
                    hs.append(pltpu.async_copy(bufs[k],
                                               acc.at[didx2.at[i + k]],
                                               ssem[k], add=True))
                for h in hs:
                    h.wait()

            plsc.subcore_barrier()
            out_idx = c if per_sc_edges_split else c * passes + q
            pltpu.sync_copy(acc.at[rows], agg_h.at[out_idx, rows])

    return functools.partial(
        pl.kernel,
        out_type=jax.ShapeDtypeStruct((nout, NP, 64), jnp.float32),
        mesh=_MESH,
        compiler_params=_SC_PARAMS,
        scratch_types=(
            [pltpu.VMEM((nrows, chunk), jnp.int32),
             pltpu.VMEM((nrows, chunk), jnp.int32)]
            + [pltpu.VMEM((chunk, 64), jnp.float32)] * nbuf
            + [pltpu.VMEM_SHARED((NP, 64), jnp.float32),
               pltpu.VMEM_SHARED((NP, 64), jnp.float32)]
            + [pltpu.SemaphoreType.DMA] * (2 * nbuf)
        ),
    )(body)


_agg1_call = _make_agg(passes=2, per_sc_edges_split=False, nbuf=5)
_agg2_call = _make_agg(passes=1, per_sc_edges_split=True, nbuf=5)


_PRED_NBUF = 5
_PRED_ROWS = NEDGE // (NC * NS) // CHUNK


def _pred_body(pe_h, ne_h, a_h, b_h, w_h, sp_h, sn_h, uidx2, vidx2, *rest):
    c = lax.axis_index("c")
    s = lax.axis_index("s")
    nbuf = _PRED_NBUF
    bufs = rest[:nbuf]
    obufs = rest[nbuf:2 * nbuf]
    wv = rest[2 * nbuf]
    sha = rest[2 * nbuf + 1]
    shb = rest[2 * nbuf + 2]
    asem = rest[2 * nbuf + 3:3 * nbuf + 3]
    bsem = rest[3 * nbuf + 3:4 * nbuf + 3]
    osem = rest[4 * nbuf + 3:5 * nbuf + 3]
    rbase = (c * NS + s) * _PRED_ROWS
    rows = pl.ds(s * RP, RP)
    pltpu.sync_copy(a_h.at[rows], sha.at[rows])
    pltpu.sync_copy(b_h.at[rows], shb.at[rows])
    pltpu.sync_copy(w_h, wv)
    w0 = wv[pl.ds(0, 16)]
    w1 = wv[pl.ds(16, 16)]
    w2 = wv[pl.ds(32, 16)]
    w3 = wv[pl.ds(48, 16)]
    bp2s = wv[pl.ds(64, 16)][0]
    lane = lax.iota(jnp.int32, 16)
    plsc.subcore_barrier()

    for e_h, o_h in ((pe_h, sp_h), (ne_h, sn_h)):
        pltpu.sync_copy(e_h.at[0, pl.ds(rbase, _PRED_ROWS)], uidx2)
        pltpu.sync_copy(e_h.at[1, pl.ds(rbase, _PRED_ROWS)], vidx2)

        @pl.loop(0, _PRED_ROWS, step=nbuf)
        def _grp(i):
            ha = [pltpu.async_copy(sha.at[uidx2.at[i + k]],
                                   bufs[k].at[pl.ds(0, CHUNK)], asem[k])
                  for k in range(nbuf)]
            hb = []
            for k in range(nbuf):
                ha[k].wait()
                hb.append(pltpu.async_copy(shb.at[vidx2.at[i + k]],
                                           bufs[k].at[pl.ds(0, CHUNK)],
                                           bsem[k], add=True))
            ho = []
            for k in range(nbuf):
                hb[k].wait()
                buf = bufs[k]
                obuf = obufs[k]

                @pl.loop(0, 3)
                def _egrp(g):
                    z = jnp.float32(0.0)
                    svec = jnp.zeros((16,), jnp.float32)
                    for j in range(16):
                        e = g * 16 + j
                        acc = (jnp.maximum(buf[e, 0:16], z) * w0
                               + jnp.maximum(buf[e, 16:32], z) * w1
                               + jnp.maximum(buf[e, 32:48], z) * w2
                               + jnp.maximum(buf[e, 48:64], z) * w3)
                        sval = jnp.sum(acc) + bp2s
                        svec = jnp.where(lane == j, sval, svec)
                    obuf[pl.ds(g * 16, 16)] = svec

                ho.append(pltpu.async_copy(
                    obuf.at[pl.ds(0, CHUNK)],
                    o_h.at[pl.ds((rbase + i + k) * CHUNK, CHUNK)],
                    osem[k]))
            for h in ho:
                h.wait()


_pred_call = functools.partial(
    pl.kernel,
    out_type=[jax.ShapeDtypeStruct((NEDGE,), jnp.float32),
              jax.ShapeDtypeStruct((NEDGE,), jnp.float32)],
    mesh=_MESH,
    compiler_params=pltpu.CompilerParams(use_tc_tiling_on_sc=False,
                                         needs_layout_passes=False),
    scratch_types=(
        [pltpu.VMEM((_PRED_ROWS, CHUNK), jnp.int32),
         pltpu.VMEM((_PRED_ROWS, CHUNK), jnp.int32)]
        + [pltpu.VMEM((48, 64), jnp.float32)] * _PRED_NBUF
        + [pltpu.VMEM((48,), jnp.float32)] * _PRED_NBUF
        + [pltpu.VMEM((80,), jnp.float32),
           pltpu.VMEM_SHARED((NP, 64), jnp.float32),
           pltpu.VMEM_SHARED((NP, 64), jnp.float32)]
        + [pltpu.SemaphoreType.DMA] * (3 * _PRED_NBUF)
    ),
)(_pred_body)


_RB = 1000


def _rsqrt_deg(ref):
    d = ref[0, :, 0:1] + ref[1, :, 0:1]
    return lax.rsqrt(jnp.maximum(d, 1.0))


def _k2_body(x_ref, w_ref, dego_ref, out_ref):
    ns = _rsqrt_deg(dego_ref)
    xw = jnp.dot(x_ref[...].astype(jnp.bfloat16),
                 w_ref[0].astype(jnp.bfloat16),
                 preferred_element_type=jnp.float32)
    out_ref[0] = xw * ns


def _k2(x, W1q, degout_p):
    return pl.pallas_call(
        _k2_body,
        grid=(10, 4),
        in_specs=[
            pl.BlockSpec((_RB, 512), lambda i, h: (i, 0)),
            pl.BlockSpec((1, 512, 64), lambda i, h: (h, 0, 0)),
            pl.BlockSpec((2, _RB, 16), lambda i, h: (0, i, 0)),
        ],
        out_specs=pl.BlockSpec((1, _RB, 64), lambda i, h: (h, i, 0)),
        out_shape=jax.ShapeDtypeStruct((4, NP, 64), jnp.float32),
    )(x, W1q, degout_p)


def _k4_body(agg_ref, dego_ref, degi_ref, b1_ref, w2_ref, out_ref):
    ns = _rsqrt_deg(dego_ref)
    nd = _rsqrt_deg(degi_ref)
    t = None
    for q in range(4):
        a = jnp.maximum(agg_ref[q] * nd + b1_ref[q:q + 1, :], 0.0)
        aq = jnp.dot(a, w2_ref[64 * q:64 * q + 64],
                     preferred_element_type=jnp.float32)
        t = aq if t is None else t + aq
    out_ref[...] = t * ns


def _k4(agg1_p, degout_p, degin_p, b1r, W2):
    return pl.pallas_call(
        _k4_body,
        grid=(10,),
        in_specs=[
            pl.BlockSpec((4, _RB, 64), lambda i: (0, i, 0)),
            pl.BlockSpec((2, _RB, 16), lambda i: (0, i, 0)),
            pl.BlockSpec((2, _RB, 16), lambda i: (0, i, 0)),
            pl.BlockSpec((4, 64), lambda i: (0, 0)),
            pl.BlockSpec((256, 64), lambda i: (0, 0)),
        ],
        out_specs=pl.BlockSpec((_RB, 64), lambda i: (i, 0)),
        out_shape=jax.ShapeDtypeStruct((NP, 64), jnp.float32),
    )(agg1_p, degout_p, degin_p, b1r, W2)


def _k6_body(agg_ref, degi_ref, b2_ref, wp1_ref, bp1_ref, a_ref, b_ref):
    nd = _rsqrt_deg(degi_ref)
    h2 = (agg_ref[0] + agg_ref[1]) * nd + b2_ref[0:1, :]
    a_ref[...] = (jnp.dot(h2, wp1_ref[0:64], preferred_element_type=jnp.float32)
                  + bp1_ref[0:1, :])
    b_ref[...] = jnp.dot(h2, wp1_ref[64:128], preferred_element_type=jnp.float32)


def _k6(agg2_p, degin_p, b2r, Wp1, bp1r):
    return pl.pallas_call(
        _k6_body,
        grid=(10,),
        in_specs=[
            pl.BlockSpec((2, _RB, 64), lambda i: (0, i, 0)),
            pl.BlockSpec((2, _RB, 16), lambda i: (0, i, 0)),
            pl.BlockSpec((1, 64), lambda i: (0, 0)),
            pl.BlockSpec((128, 64), lambda i: (0, 0)),
            pl.BlockSpec((1, 64), lambda i: (0, 0)),
        ],
        out_specs=[
            pl.BlockSpec((_RB, 64), lambda i: (i, 0)),
            pl.BlockSpec((_RB, 64), lambda i: (i, 0)),
        ],
        out_shape=[jax.ShapeDtypeStruct((NP, 64), jnp.float32),
                   jax.ShapeDtypeStruct((NP, 64), jnp.float32)],
    )(agg2_p, degin_p, b2r, Wp1, bp1r)


def kernel(x, edge_index, pos_edge_index, neg_edge_index,
           W1, b1, W2, b2, Wp1, bp1, Wp2, bp2):
    ones16 = jnp.ones((CHUNK, 16), jnp.float32)
    zeros16 = jnp.zeros((RP, 16), jnp.float32)
    zeros64 = jnp.zeros((RP, 64), jnp.float32)

    e3 = edge_index.reshape(2, NEDGE // CHUNK, CHUNK)
    p3 = pos_edge_index.reshape(2, NEDGE // CHUNK, CHUNK)
    n3 = neg_edge_index.reshape(2, NEDGE // CHUNK, CHUNK)

    degout_p, degin_p = _deg_call(e3, ones16, zeros16)

    w1q = W1.reshape(512, 4, 64).transpose(1, 0, 2)
    h1s = _k2(x, w1q, degout_p)
    agg1_p = _agg1_call(e3, h1s, zeros64)

    h2in = _k4(agg1_p, degout_p, degin_p, b1.reshape(4, 64), W2)
    agg2_p = _agg2_call(e3, h2in.reshape(1, NP, 64), zeros64)

    A, B = _k6(agg2_p, degin_p, b2.reshape(1, 64), Wp1, bp1.reshape(1, 64))
    wvec = jnp.concatenate([Wp2.reshape(64), bp2, jnp.zeros((15,), jnp.float32)])
    pos, neg = _pred_call(p3, n3, A, B, wvec)
    return (pos, neg)


# speedup vs baseline: 1.2092x; 1.1765x over previous
import functools

import jax
import jax.numpy as jnp
from jax import lax
from jax.experimental import pallas as pl
from jax.experimental.pallas import tpu as pltpu
from jax.experimental.pallas import tpu_sc as plsc

NNODE = 10000
NP = 10240
NEDGE = 160000
CHUNK = 40
NC, NS = 2, 16
RP = NP // NS

_MESH = plsc.VectorSubcoreMesh(core_axis_name="c", subcore_axis_name="s")
_SC_PARAMS = pltpu.CompilerParams(use_tc_tiling_on_sc=False)


_DEG_ROWS = NEDGE // (NC * NS) // CHUNK


def _deg_body(e_h, ones_h, zeros_h, outp_h, inp_h,
              sidx2, didx2, ones_v, shout, shin, sem_a, sem_b):
    c = lax.axis_index("c")
    s = lax.axis_index("s")
    rbase = (c * NS + s) * _DEG_ROWS
    pltpu.sync_copy(e_h.at[0, pl.ds(rbase, _DEG_ROWS)], sidx2)
    pltpu.sync_copy(e_h.at[1, pl.ds(rbase, _DEG_ROWS)], didx2)
    pltpu.sync_copy(ones_h, ones_v)
    rows = pl.ds(s * RP, RP)
    pltpu.sync_copy(zeros_h, shout.at[rows])
    pltpu.sync_copy(zeros_h, shin.at[rows])
    plsc.subcore_barrier()

    @pl.loop(0, _DEG_ROWS, step=5)
    def _grp(i):
        hs = []
        for k in range(5):
            hs.append(pltpu.async_copy(ones_v, shout.at[sidx2.at[i + k]],
                                       sem_a, add=True))
            hs.append(pltpu.async_copy(ones_v, shin.at[didx2.at[i + k]],
                                       sem_b, add=True))
        for h in hs:
            h.wait()

    plsc.subcore_barrier()
    pltpu.sync_copy(shout.at[rows], outp_h.at[c, rows])
    pltpu.sync_copy(shin.at[rows], inp_h.at[c, rows])


_deg_call = functools.partial(
    pl.kernel,
    out_type=[jax.ShapeDtypeStruct((NC, NP, 16), jnp.float32),
              jax.ShapeDtypeStruct((NC, NP, 16), jnp.float32)],
    mesh=_MESH,
    compiler_params=_SC_PARAMS,
    scratch_types=[
        pltpu.VMEM((_DEG_ROWS, CHUNK), jnp.int32),
        pltpu.VMEM((_DEG_ROWS, CHUNK), jnp.int32),
        pltpu.VMEM((CHUNK, 16), jnp.float32),
        pltpu.VMEM_SHARED((NP, 16), jnp.float32),
        pltpu.VMEM_SHARED((NP, 16), jnp.float32),
        pltpu.SemaphoreType.DMA,
        pltpu.SemaphoreType.DMA,
    ],
)(_deg_body)


def _make_agg(passes, per_sc_edges_split, nbuf, chunk=CHUNK):
    if per_sc_edges_split:
        nrows = NEDGE // (NC * NS) // chunk
    else:
        nrows = NEDGE // NS // chunk
    out_shape = ((NP, 128) if per_sc_edges_split
                 else (NC, NP, 128))

    def body(e_h, h_h, zeros_h, agg_h, sidx2, didx2, *rest):
        bufs = rest[:nbuf]
        table = rest[nbuf]
        acc = rest[nbuf + 1]
        gsem = rest[nbuf + 2:2 * nbuf + 2]
        ssem = rest[2 * nbuf + 2:3 * nbuf + 2]
        c = lax.axis_index("c")
        s = lax.axis_index("s")
        if per_sc_edges_split:
            rbase = (c * NS + s) * nrows
        else:
            rbase = s * nrows
        rows = pl.ds(s * RP, RP)
        pltpu.sync_copy(e_h.at[0, pl.ds(rbase, nrows)], sidx2)
        pltpu.sync_copy(e_h.at[1, pl.ds(rbase, nrows)], didx2)

        for q in range(passes):
            if per_sc_edges_split:
                pltpu.sync_copy(h_h.at[rows, pl.ds(0, 64)], table.at[rows])
            else:
                pltpu.sync_copy(h_h.at[c, rows, pl.ds(64 * q, 64)],
                                table.at[rows])
            pltpu.sync_copy(zeros_h, acc.at[rows])
            plsc.subcore_barrier()

            @pl.loop(0, nrows, step=nbuf)
            def _grp(i):
                hg = [pltpu.async_copy(table.at[sidx2.at[i + k]], bufs[k],
                                       gsem[k]) for k in range(nbuf)]
                hs = []
                for k in range(nbuf):
                    hg[k].wait()
                    hs.append(pltpu.async_copy(bufs[k],
                                               acc.at[didx2.at[i + k]],
                                               ssem[k], add=True))
                for h in hs:
                    h.wait()

            plsc.subcore_barrier()
            if per_sc_edges_split:
                pltpu.sync_copy(acc.at[rows],
                                agg_h.at[rows, pl.ds(64 * c, 64)])
            else:
                pltpu.sync_copy(acc.at[rows],
                                agg_h.at[c, rows, pl.ds(64 * q, 64)])

    return functools.partial(
        pl.kernel,
        out_type=jax.ShapeDtypeStruct(out_shape, jnp.float32),
        mesh=_MESH,
        compiler_params=_SC_PARAMS,
        scratch_types=(
            [pltpu.VMEM((nrows, chunk), jnp.int32),
             pltpu.VMEM((nrows, chunk), jnp.int32)]
            + [pltpu.VMEM((chunk, 64), jnp.float32)] * nbuf
            + [pltpu.VMEM_SHARED((NP, 64), jnp.float32),
               pltpu.VMEM_SHARED((NP, 64), jnp.float32)]
            + [pltpu.SemaphoreType.DMA] * (2 * nbuf)
        ),
    )(body)


_agg1_call = _make_agg(passes=2, per_sc_edges_split=False, nbuf=5)
_agg2_call = _make_agg(passes=1, per_sc_edges_split=True, nbuf=5)


_PRED_NBUF = 5
_PRED_ROWS = NEDGE // (NC * NS) // CHUNK


def _pred_body(pe_h, ne_h, ab_h, w_h, sp_h, sn_h, uidx2, vidx2, *rest):
    c = lax.axis_index("c")
    s = lax.axis_index("s")
    nbuf = _PRED_NBUF
    bufs = rest[:nbuf]
    obufs = rest[nbuf:2 * nbuf]
    wv = rest[2 * nbuf]
    sha = rest[2 * nbuf + 1]
    shb = rest[2 * nbuf + 2]
    asem = rest[2 * nbuf + 3:3 * nbuf + 3]
    bsem = rest[3 * nbuf + 3:4 * nbuf + 3]
    osem = rest[4 * nbuf + 3:5 * nbuf + 3]
    rbase = (c * NS + s) * _PRED_ROWS
    rows = pl.ds(s * RP, RP)
    pltpu.sync_copy(ab_h.at[rows, pl.ds(0, 64)], sha.at[rows])
    pltpu.sync_copy(ab_h.at[rows, pl.ds(64, 64)], shb.at[rows])
    pltpu.sync_copy(w_h, wv)
    w0 = wv[pl.ds(0, 16)]
    w1 = wv[pl.ds(16, 16)]
    w2 = wv[pl.ds(32, 16)]
    w3 = wv[pl.ds(48, 16)]
    bp2s = wv[pl.ds(64, 16)][0]
    lane = lax.iota(jnp.int32, 16)
    plsc.subcore_barrier()

    for e_h, o_h in ((pe_h, sp_h), (ne_h, sn_h)):
        pltpu.sync_copy(e_h.at[0, pl.ds(rbase, _PRED_ROWS)], uidx2)
        pltpu.sync_copy(e_h.at[1, pl.ds(rbase, _PRED_ROWS)], vidx2)

        @pl.loop(0, _PRED_ROWS, step=nbuf)
        def _grp(i):
            ha = [pltpu.async_copy(sha.at[uidx2.at[i + k]],
                                   bufs[k].at[pl.ds(0, CHUNK)], asem[k])
                  for k in range(nbuf)]
            hb = []
            for k in range(nbuf):
                ha[k].wait()
                hb.append(pltpu.async_copy(shb.at[vidx2.at[i + k]],
                                           bufs[k].at[pl.ds(0, CHUNK)],
                                           bsem[k], add=True))
            ho = []
            for k in range(nbuf):
                hb[k].wait()
                buf = bufs[k]
                obuf = obufs[k]

                @pl.loop(0, 3)
                def _egrp(g):
                    z = jnp.float32(0.0)
                    svec = jnp.zeros((16,), jnp.float32)
                    for j in range(16):
                        e = g * 16 + j
                        acc = (jnp.maximum(buf[e, 0:16], z) * w0
                               + jnp.maximum(buf[e, 16:32], z) * w1
                               + jnp.maximum(buf[e, 32:48], z) * w2
                               + jnp.maximum(buf[e, 48:64], z) * w3)
                        sval = jnp.sum(acc) + bp2s
                        svec = jnp.where(lane == j, sval, svec)
                    obuf[pl.ds(g * 16, 16)] = svec

                ho.append(pltpu.async_copy(
                    obuf.at[pl.ds(0, CHUNK)],
                    o_h.at[pl.ds((rbase + i + k) * CHUNK, CHUNK)],
                    osem[k]))
            for h in ho:
                h.wait()


_pred_call = functools.partial(
    pl.kernel,
    out_type=[jax.ShapeDtypeStruct((NEDGE,), jnp.float32),
              jax.ShapeDtypeStruct((NEDGE,), jnp.float32)],
    mesh=_MESH,
    compiler_params=pltpu.CompilerParams(use_tc_tiling_on_sc=False,
                                         needs_layout_passes=False),
    scratch_types=(
        [pltpu.VMEM((_PRED_ROWS, CHUNK), jnp.int32),
         pltpu.VMEM((_PRED_ROWS, CHUNK), jnp.int32)]
        + [pltpu.VMEM((48, 64), jnp.float32)] * _PRED_NBUF
        + [pltpu.VMEM((48,), jnp.float32)] * _PRED_NBUF
        + [pltpu.VMEM((80,), jnp.float32),
           pltpu.VMEM_SHARED((NP, 64), jnp.float32),
           pltpu.VMEM_SHARED((NP, 64), jnp.float32)]
        + [pltpu.SemaphoreType.DMA] * (3 * _PRED_NBUF)
    ),
)(_pred_body)


_RB = 1000


def _rsqrt_deg(ref):
    d = ref[0, :, 0:1] + ref[1, :, 0:1]
    return lax.rsqrt(jnp.maximum(d, 1.0))


def _k2_body(x_ref, w_ref, dego_ref, out_ref):
    ns = _rsqrt_deg(dego_ref)
    xw = jnp.dot(x_ref[...], w_ref[0], preferred_element_type=jnp.float32)
    out_ref[0] = xw * ns


def _k2(x, W1h, degout_p):
    return pl.pallas_call(
        _k2_body,
        grid=(10, 2),
        in_specs=[
            pl.BlockSpec((_RB, 512), lambda i, h: (i, 0)),
            pl.BlockSpec((1, 512, 128), lambda i, h: (h, 0, 0)),
            pl.BlockSpec((2, _RB, 16), lambda i, h: (0, i, 0)),
        ],
        out_specs=pl.BlockSpec((1, _RB, 128), lambda i, h: (h, i, 0)),
        out_shape=jax.ShapeDtypeStruct((2, NP, 128), jnp.float32),
    )(x, W1h, degout_p)


def _k4_body(agg_ref, dego_ref, degi_ref, b1_ref, w2_ref, out_ref):
    ns = _rsqrt_deg(dego_ref)
    nd = _rsqrt_deg(degi_ref)
    t = None
    for q in range(4):
        h, j = divmod(q, 2)
        a = jnp.maximum(agg_ref[h][:, 64 * j:64 * j + 64] * nd
                        + b1_ref[q:q + 1, :], 0.0)
        aq = jnp.dot(a, w2_ref[64 * q:64 * q + 64],
                     preferred_element_type=jnp.float32)
        t = aq if t is None else t + aq
    t = t * ns
    out_ref[...] = jnp.concatenate([t, jnp.zeros_like(t)], axis=1)


def _k4(agg1_p, degout_p, degin_p, b1r, W2):
    return pl.pallas_call(
        _k4_body,
        grid=(10,),
        in_specs=[
            pl.BlockSpec((2, _RB, 128), lambda i: (0, i, 0)),
            pl.BlockSpec((2, _RB, 16), lambda i: (0, i, 0)),
            pl.BlockSpec((2, _RB, 16), lambda i: (0, i, 0)),
            pl.BlockSpec((4, 64), lambda i: (0, 0)),
            pl.BlockSpec((256, 64), lambda i: (0, 0)),
        ],
        out_specs=pl.BlockSpec((_RB, 128), lambda i: (i, 0)),
        out_shape=jax.ShapeDtypeStruct((NP, 128), jnp.float32),
    )(agg1_p, degout_p, degin_p, b1r, W2)


def _k6_body(agg_ref, degi_ref, b2_ref, wp1_ref, bp1_ref, ab_ref):
    nd = _rsqrt_deg(degi_ref)
    g = agg_ref[...]
    h2 = (g[:, 0:64] + g[:, 64:128]) * nd + b2_ref[0:1, :]
    a = (jnp.dot(h2, wp1_ref[0:64], preferred_element_type=jnp.float32)
         + bp1_ref[0:1, :])
    b = jnp.dot(h2, wp1_ref[64:128], preferred_element_type=jnp.float32)
    ab_ref[...] = jnp.concatenate([a, b], axis=1)


def _k6(agg2_p, degin_p, b2r, Wp1, bp1r):
    return pl.pallas_call(
        _k6_body,
        grid=(10,),
        in_specs=[
            pl.BlockSpec((_RB, 128), lambda i: (i, 0)),
            pl.BlockSpec((2, _RB, 16), lambda i: (0, i, 0)),
            pl.BlockSpec((1, 64), lambda i: (0, 0)),
            pl.BlockSpec((128, 64), lambda i: (0, 0)),
            pl.BlockSpec((1, 64), lambda i: (0, 0)),
        ],
        out_specs=pl.BlockSpec((_RB, 128), lambda i: (i, 0)),
        out_shape=jax.ShapeDtypeStruct((NP, 128), jnp.float32),
    )(agg2_p, degin_p, b2r, Wp1, bp1r)


def kernel(x, edge_index, pos_edge_index, neg_edge_index,
           W1, b1, W2, b2, Wp1, bp1, Wp2, bp2):
    ones16 = jnp.ones((CHUNK, 16), jnp.float32)
    zeros16 = jnp.zeros((RP, 16), jnp.float32)
    zeros64 = jnp.zeros((RP, 64), jnp.float32)

    e3 = edge_index.reshape(2, NEDGE // CHUNK, CHUNK)
    p3 = pos_edge_index.reshape(2, NEDGE // CHUNK, CHUNK)
    n3 = neg_edge_index.reshape(2, NEDGE // CHUNK, CHUNK)

    degout_p, degin_p = _deg_call(e3, ones16, zeros16)

    w1h = W1.reshape(512, 2, 128).transpose(1, 0, 2)
    h1s = _k2(x, w1h, degout_p)
    agg1_p = _agg1_call(e3, h1s, zeros64)

    h2in = _k4(agg1_p, degout_p, degin_p, b1.reshape(4, 64), W2)
    agg2_p = _agg2_call(e3, h2in, zeros64)

    AB = _k6(agg2_p, degin_p, b2.reshape(1, 64), Wp1, bp1.reshape(1, 64))
    wvec = jnp.concatenate([Wp2.reshape(64), bp2, jnp.zeros((15,), jnp.float32)])
    pos, neg = _pred_call(p3, n3, AB, wvec)
    return (pos, neg)


# speedup vs baseline: 1.2392x; 1.0248x over previous
import functools

import jax
import jax.numpy as jnp
from jax import lax
from jax.experimental import pallas as pl
from jax.experimental.pallas import tpu as pltpu
from jax.experimental.pallas import tpu_sc as plsc

NNODE = 10000
NP = 10240
NEDGE = 160000
CHUNK = 40
NC, NS = 2, 16
RP = NP // NS

_MESH = plsc.VectorSubcoreMesh(core_axis_name="c", subcore_axis_name="s")
_SC_PARAMS = pltpu.CompilerParams(use_tc_tiling_on_sc=False)


_DEG_ROWS = NEDGE // (NC * NS) // CHUNK


def _deg_body(e_h, ones_h, zeros_h, outp_h, inp_h,
              sidx2, didx2, ones_v, shout, shin, sem_a, sem_b):
    c = lax.axis_index("c")
    s = lax.axis_index("s")
    rbase = (c * NS + s) * _DEG_ROWS
    pltpu.sync_copy(e_h.at[0, pl.ds(rbase, _DEG_ROWS)], sidx2)
    pltpu.sync_copy(e_h.at[1, pl.ds(rbase, _DEG_ROWS)], didx2)
    pltpu.sync_copy(ones_h, ones_v)
    rows = pl.ds(s * RP, RP)
    pltpu.sync_copy(zeros_h, shout.at[rows])
    pltpu.sync_copy(zeros_h, shin.at[rows])
    plsc.subcore_barrier()

    @pl.loop(0, _DEG_ROWS, step=5)
    def _grp(i):
        hs = []
        for k in range(5):
            hs.append(pltpu.async_copy(ones_v, shout.at[sidx2.at[i + k]],
                                       sem_a, add=True))
            hs.append(pltpu.async_copy(ones_v, shin.at[didx2.at[i + k]],
                                       sem_b, add=True))
        for h in hs:
            h.wait()

    plsc.subcore_barrier()
    pltpu.sync_copy(shout.at[rows], outp_h.at[c, rows])
    pltpu.sync_copy(shin.at[rows], inp_h.at[c, rows])


_deg_call = functools.partial(
    pl.kernel,
    out_type=[jax.ShapeDtypeStruct((NC, NP, 16), jnp.float32),
              jax.ShapeDtypeStruct((NC, NP, 16), jnp.float32)],
    mesh=_MESH,
    compiler_params=_SC_PARAMS,
    scratch_types=[
        pltpu.VMEM((_DEG_ROWS, CHUNK), jnp.int32),
        pltpu.VMEM((_DEG_ROWS, CHUNK), jnp.int32),
        pltpu.VMEM((CHUNK, 16), jnp.float32),
        pltpu.VMEM_SHARED((NP, 16), jnp.float32),
        pltpu.VMEM_SHARED((NP, 16), jnp.float32),
        pltpu.SemaphoreType.DMA,
        pltpu.SemaphoreType.DMA,
    ],
)(_deg_body)


def _make_agg(passes, per_sc_edges_split, nbuf, chunk=CHUNK):
    if per_sc_edges_split:
        nrows = NEDGE // (NC * NS) // chunk
    else:
        nrows = NEDGE // NS // chunk
    out_shape = ((NP, 128) if per_sc_edges_split
                 else (NC, NP, 128))

    def body(e_h, h_h, zeros_h, agg_h, sidx2, didx2, *rest):
        bufs = rest[:nbuf]
        table = rest[nbuf]
        acc = rest[nbuf + 1]
        gsem = rest[nbuf + 2:2 * nbuf + 2]
        ssem = rest[2 * nbuf + 2:3 * nbuf + 2]
        c = lax.axis_index("c")
        s = lax.axis_index("s")
        if per_sc_edges_split:
            rbase = (c * NS + s) * nrows
        else:
            rbase = s * nrows
        rows = pl.ds(s * RP, RP)
        pltpu.sync_copy(e_h.at[0, pl.ds(rbase, nrows)], sidx2)
        pltpu.sync_copy(e_h.at[1, pl.ds(rbase, nrows)], didx2)

        for q in range(passes):
            if per_sc_edges_split:
                pltpu.sync_copy(h_h.at[rows, pl.ds(0, 64)], table.at[rows])
            else:
                pltpu.sync_copy(h_h.at[c, rows, pl.ds(64 * q, 64)],
                                table.at[rows])
            pltpu.sync_copy(zeros_h, acc.at[rows])
            plsc.subcore_barrier()

            @pl.loop(0, nrows, step=nbuf)
            def _grp(i):
                hg = [pltpu.async_copy(table.at[sidx2.at[i + k]], bufs[k],
                                       gsem[k]) for k in range(nbuf)]
                hs = []
                for k in range(nbuf):
                    hg[k].wait()
                    hs.append(pltpu.async_copy(bufs[k],
                                               acc.at[didx2.at[i + k]],
                                               ssem[k], add=True))
                for h in hs:
                    h.wait()

            plsc.subcore_barrier()
            if per_sc_edges_split:
                pltpu.sync_copy(acc.at[rows],
                                agg_h.at[rows, pl.ds(64 * c, 64)])
            else:
                pltpu.sync_copy(acc.at[rows],
                                agg_h.at[c, rows, pl.ds(64 * q, 64)])

    return functools.partial(
        pl.kernel,
        out_type=jax.ShapeDtypeStruct(out_shape, jnp.float32),
        mesh=_MESH,
        compiler_params=_SC_PARAMS,
        scratch_types=(
            [pltpu.VMEM((nrows, chunk), jnp.int32),
             pltpu.VMEM((nrows, chunk), jnp.int32)]
            + [pltpu.VMEM((chunk, 64), jnp.float32)] * nbuf
            + [pltpu.VMEM_SHARED((NP, 64), jnp.float32),
               pltpu.VMEM_SHARED((NP, 64), jnp.float32)]
            + [pltpu.SemaphoreType.DMA] * (2 * nbuf)
        ),
    )(body)


_agg1_call = _make_agg(passes=2, per_sc_edges_split=False, nbuf=10)
_agg2_call = _make_agg(passes=1, per_sc_edges_split=True, nbuf=5)


_PRED_NBUF = 5
_PRED_ROWS = NEDGE // (NC * NS) // CHUNK


def _pred_body(pe_h, ne_h, ab_h, w_h, sp_h, sn_h, uidx2, vidx2, *rest):
    c = lax.axis_index("c")
    s = lax.axis_index("s")
    nbuf = _PRED_NBUF
    bufs = rest[:nbuf]
    obuf = rest[nbuf]
    wv = rest[nbuf + 1]
    sha = rest[nbuf + 2]
    shb = rest[nbuf + 3]
    asem = rest[nbuf + 4:2 * nbuf + 4]
    bsem = rest[2 * nbuf + 4:3 * nbuf + 4]
    osem = rest[3 * nbuf + 4]
    rbase = (c * NS + s) * _PRED_ROWS
    rows = pl.ds(s * RP, RP)
    pltpu.sync_copy(ab_h.at[rows, pl.ds(0, 64)], sha.at[rows])
    pltpu.sync_copy(ab_h.at[rows, pl.ds(64, 64)], shb.at[rows])
    pltpu.sync_copy(w_h, wv)
    w0 = wv[pl.ds(0, 16)]
    w1 = wv[pl.ds(16, 16)]
    w2 = wv[pl.ds(32, 16)]
    w3 = wv[pl.ds(48, 16)]
    bp2s = wv[pl.ds(64, 16)][0]
    lane = lax.iota(jnp.int32, 16)
    plsc.subcore_barrier()

    for e_h, o_h in ((pe_h, sp_h), (ne_h, sn_h)):
        pltpu.sync_copy(e_h.at[0, pl.ds(rbase, _PRED_ROWS)], uidx2)
        pltpu.sync_copy(e_h.at[1, pl.ds(rbase, _PRED_ROWS)], vidx2)

        @pl.loop(0, _PRED_ROWS, step=nbuf)
        def _grp(i):
            ha = [pltpu.async_copy(sha.at[uidx2.at[i + k]],
                                   bufs[k].at[pl.ds(0, CHUNK)], asem[k])
                  for k in range(nbuf)]
            hb = []
            for k in range(nbuf):
                ha[k].wait()
                hb.append(pltpu.async_copy(shb.at[vidx2.at[i + k]],
                                           bufs[k].at[pl.ds(0, CHUNK)],
                                           bsem[k], add=True))
            for k in range(nbuf):
                hb[k].wait()
                buf = bufs[k]

                @pl.loop(0, 3)
                def _egrp(g):
                    z = jnp.float32(0.0)
                    svec = jnp.zeros((16,), jnp.float32)
                    for j in range(16):
                        e = g * 16 + j
                        acc = (jnp.maximum(buf[e, 0:16], z) * w0
                               + jnp.maximum(buf[e, 16:32], z) * w1
                               + jnp.maximum(buf[e, 32:48], z) * w2
                               + jnp.maximum(buf[e, 48:64], z) * w3)
                        sval = jnp.sum(acc) + bp2s
                        svec = jnp.where(lane == j, sval, svec)
                    off = (i + k) * CHUNK + g * 16
                    obuf[pl.ds(off, 16)] = svec

        pltpu.sync_copy(obuf.at[pl.ds(0, _PRED_ROWS * CHUNK)],
                        o_h.at[pl.ds(rbase * CHUNK, _PRED_ROWS * CHUNK)])


_pred_call = functools.partial(
    pl.kernel,
    out_type=[jax.ShapeDtypeStruct((NEDGE,), jnp.float32),
              jax.ShapeDtypeStruct((NEDGE,), jnp.float32)],
    mesh=_MESH,
    compiler_params=pltpu.CompilerParams(use_tc_tiling_on_sc=False,
                                         needs_layout_passes=False),
    scratch_types=(
        [pltpu.VMEM((_PRED_ROWS, CHUNK), jnp.int32),
         pltpu.VMEM((_PRED_ROWS, CHUNK), jnp.int32)]
        + [pltpu.VMEM((48, 64), jnp.float32)] * _PRED_NBUF
        + [pltpu.VMEM((_PRED_ROWS * CHUNK + 16,), jnp.float32),
           pltpu.VMEM((80,), jnp.float32),
           pltpu.VMEM_SHARED((NP, 64), jnp.float32),
           pltpu.VMEM_SHARED((NP, 64), jnp.float32)]
        + [pltpu.SemaphoreType.DMA] * (2 * _PRED_NBUF + 1)
    ),
)(_pred_body)


_RB = 1000


def _rsqrt_deg(ref):
    d = ref[0, :, 0:1] + ref[1, :, 0:1]
    return lax.rsqrt(jnp.maximum(d, 1.0))


def _k2_body(x_ref, w_ref, dego_ref, out_ref):
    ns = _rsqrt_deg(dego_ref)
    xw = jnp.dot(x_ref[...], w_ref[0], preferred_element_type=jnp.float32)
    out_ref[0] = xw * ns


def _k2(x, W1h, degout_p):
    return pl.pallas_call(
        _k2_body,
        grid=(10, 2),
        in_specs=[
            pl.BlockSpec((_RB, 512), lambda i, h: (i, 0)),
            pl.BlockSpec((1, 512, 128), lambda i, h: (h, 0, 0)),
            pl.BlockSpec((2, _RB, 16), lambda i, h: (0, i, 0)),
        ],
        out_specs=pl.BlockSpec((1, _RB, 128), lambda i, h: (h, i, 0)),
        out_shape=jax.ShapeDtypeStruct((2, NP, 128), jnp.float32),
    )(x, W1h, degout_p)


def _k4_body(agg_ref, dego_ref, degi_ref, b1_ref, w2_ref, out_ref):
    ns = _rsqrt_deg(dego_ref)
    nd = _rsqrt_deg(degi_ref)
    t = None
    for q in range(4):
        h, j = divmod(q, 2)
        a = jnp.maximum(agg_ref[h][:, 64 * j:64 * j + 64] * nd
                        + b1_ref[q:q + 1, :], 0.0)
        aq = jnp.dot(a, w2_ref[64 * q:64 * q + 64],
                     preferred_element_type=jnp.float32)
        t = aq if t is None else t + aq
    t = t * ns
    out_ref[...] = jnp.concatenate([t, jnp.zeros_like(t)], axis=1)


def _k4(agg1_p, degout_p, degin_p, b1r, W2):
    return pl.pallas_call(
        _k4_body,
        grid=(10,),
        in_specs=[
            pl.BlockSpec((2, _RB, 128), lambda i: (0, i, 0)),
            pl.BlockSpec((2, _RB, 16), lambda i: (0, i, 0)),
            pl.BlockSpec((2, _RB, 16), lambda i: (0, i, 0)),
            pl.BlockSpec((4, 64), lambda i: (0, 0)),
            pl.BlockSpec((256, 64), lambda i: (0, 0)),
        ],
        out_specs=pl.BlockSpec((_RB, 128), lambda i: (i, 0)),
        out_shape=jax.ShapeDtypeStruct((NP, 128), jnp.float32),
    )(agg1_p, degout_p, degin_p, b1r, W2)


def _k6_body(agg_ref, degi_ref, b2_ref, wp1_ref, bp1_ref, ab_ref):
    nd = _rsqrt_deg(degi_ref)
    g = agg_ref[...]
    h2 = (g[:, 0:64] + g[:, 64:128]) * nd + b2_ref[0:1, :]
    a = (jnp.dot(h2, wp1_ref[0:64], preferred_element_type=jnp.float32)
         + bp1_ref[0:1, :])
    b = jnp.dot(h2, wp1_ref[64:128], preferred_element_type=jnp.float32)
    ab_ref[...] = jnp.concatenate([a, b], axis=1)


def _k6(agg2_p, degin_p, b2r, Wp1, bp1r):
    return pl.pallas_call(
        _k6_body,
        grid=(10,),
        in_specs=[
            pl.BlockSpec((_RB, 128), lambda i: (i, 0)),
            pl.BlockSpec((2, _RB, 16), lambda i: (0, i, 0)),
            pl.BlockSpec((1, 64), lambda i: (0, 0)),
            pl.BlockSpec((128, 64), lambda i: (0, 0)),
            pl.BlockSpec((1, 64), lambda i: (0, 0)),
        ],
        out_specs=pl.BlockSpec((_RB, 128), lambda i: (i, 0)),
        out_shape=jax.ShapeDtypeStruct((NP, 128), jnp.float32),
    )(agg2_p, degin_p, b2r, Wp1, bp1r)


def kernel(x, edge_index, pos_edge_index, neg_edge_index,
           W1, b1, W2, b2, Wp1, bp1, Wp2, bp2):
    ones16 = jnp.ones((CHUNK, 16), jnp.float32)
    zeros16 = jnp.zeros((RP, 16), jnp.float32)
    zeros64 = jnp.zeros((RP, 64), jnp.float32)

    e3 = edge_index.reshape(2, NEDGE // CHUNK, CHUNK)
    p3 = pos_edge_index.reshape(2, NEDGE // CHUNK, CHUNK)
    n3 = neg_edge_index.reshape(2, NEDGE // CHUNK, CHUNK)

    degout_p, degin_p = _deg_call(e3, ones16, zeros16)

    w1h = W1.reshape(512, 2, 128).transpose(1, 0, 2)
    h1s = _k2(x, w1h, degout_p)
    agg1_p = _agg1_call(e3, h1s, zeros64)

    h2in = _k4(agg1_p, degout_p, degin_p, b1.reshape(4, 64), W2)
    agg2_p = _agg2_call(e3, h2in, zeros64)

    AB = _k6(agg2_p, degin_p, b2.reshape(1, 64), Wp1, bp1.reshape(1, 64))
    wvec = jnp.concatenate([Wp2.reshape(64), bp2, jnp.zeros((15,), jnp.float32)])
    pos, neg = _pred_call(p3, n3, AB, wvec)
    return (pos, neg)
